# baseline (speedup 1.0000x reference)
import functools

import jax
import jax.numpy as jnp
from jax import lax
from jax.experimental import pallas as pl
from jax.experimental.pallas import tpu as pltpu
from jax.experimental.pallas import tpu_sc as plsc


def kernel(x, table):
    seq_len = x.shape[1]
    d_model = table.shape[1]

    info = plsc.get_sparse_core_info()
    num_workers = info.num_cores * info.num_subcores
    assert seq_len % num_workers == 0
    rows_per_w = seq_len // num_workers

    mesh = plsc.VectorSubcoreMesh(core_axis_name="c", subcore_axis_name="s")

    @functools.partial(
        pl.kernel,
        mesh=mesh,
        out_type=jax.ShapeDtypeStruct((seq_len, d_model), table.dtype),
    )
    def copy_rows(table_hbm, out_hbm):
        wid = lax.axis_index("s") * info.num_cores + lax.axis_index("c")
        base = wid * rows_per_w
        pltpu.sync_copy(
            table_hbm.at[pl.ds(base, rows_per_w)],
            out_hbm.at[pl.ds(base, rows_per_w)],
        )

    return copy_rows(table)[None]


# speedup vs baseline: 16.6973x; 16.6973x over previous
import functools

import jax
import jax.numpy as jnp
from jax import lax
from jax.experimental import pallas as pl
from jax.experimental.pallas import tpu as pltpu
from jax.experimental.pallas import tpu_sc as plsc

_CHUNK_ROWS = 16
_NBUF = 4


def kernel(x, table):
    seq_len = x.shape[1]
    d_model = table.shape[1]

    info = plsc.get_sparse_core_info()
    num_workers = info.num_cores * info.num_subcores
    assert seq_len % (num_workers * _CHUNK_ROWS) == 0
    rows_per_w = seq_len // num_workers
    num_chunks = rows_per_w // _CHUNK_ROWS

    mesh = plsc.VectorSubcoreMesh(core_axis_name="c", subcore_axis_name="s")

    @functools.partial(
        pl.kernel,
        mesh=mesh,
        out_type=jax.ShapeDtypeStruct((seq_len, d_model), table.dtype),
        scratch_types=[
            pltpu.VMEM((_NBUF, _CHUNK_ROWS, d_model), table.dtype),
            pltpu.SemaphoreType.DMA((_NBUF,)),
            pltpu.SemaphoreType.DMA((_NBUF,)),
        ],
    )
    def copy_rows(table_hbm, out_hbm, buf, sem_in, sem_out):
        wid = lax.axis_index("s") * info.num_cores + lax.axis_index("c")
        base = wid * rows_per_w

        def in_copy(i):
            b = i % _NBUF
            return pltpu.make_async_copy(
                table_hbm.at[pl.ds(base + i * _CHUNK_ROWS, _CHUNK_ROWS)],
                buf.at[b],
                sem_in.at[b],
            )

        def out_copy(i):
            b = i % _NBUF
            return pltpu.make_async_copy(
                buf.at[b],
                out_hbm.at[pl.ds(base + i * _CHUNK_ROWS, _CHUNK_ROWS)],
                sem_out.at[b],
            )

        in_copy(0).start()
        for i in range(num_chunks):
            if i + 1 < num_chunks:
                if i + 1 >= _NBUF:
                    out_copy(i + 1 - _NBUF).wait()
                in_copy(i + 1).start()
            in_copy(i).wait()
            out_copy(i).start()
        for i in range(max(0, num_chunks - _NBUF + 1), num_chunks):
            out_copy(i).wait()

    return copy_rows(table)[None]


# speedup vs baseline: 16.8944x; 1.0118x over previous
import functools

import jax
import jax.numpy as jnp
from jax import lax
from jax.experimental import pallas as pl
from jax.experimental.pallas import tpu as pltpu
from jax.experimental.pallas import tpu_sc as plsc

_CHUNK_ROWS = 32
_NBUF = 3


def kernel(x, table):
    seq_len = x.shape[1]
    d_model = table.shape[1]

    info = plsc.get_sparse_core_info()
    num_workers = info.num_cores * info.num_subcores
    assert seq_len % (num_workers * _CHUNK_ROWS) == 0
    rows_per_w = seq_len // num_workers
    num_chunks = rows_per_w // _CHUNK_ROWS

    mesh = plsc.VectorSubcoreMesh(core_axis_name="c", subcore_axis_name="s")

    @functools.partial(
        pl.kernel,
        mesh=mesh,
        out_type=jax.ShapeDtypeStruct((seq_len, d_model), table.dtype),
        scratch_types=[
            pltpu.VMEM((_NBUF, _CHUNK_ROWS, d_model), table.dtype),
            pltpu.SemaphoreType.DMA((_NBUF,)),
            pltpu.SemaphoreType.DMA((_NBUF,)),
        ],
    )
    def copy_rows(table_hbm, out_hbm, buf, sem_in, sem_out):
        wid = lax.axis_index("s") * info.num_cores + lax.axis_index("c")
        base = wid * rows_per_w

        def in_copy(i):
            b = i % _NBUF
            return pltpu.make_async_copy(
                table_hbm.at[pl.ds(base + i * _CHUNK_ROWS, _CHUNK_ROWS)],
                buf.at[b],
                sem_in.at[b],
            )

        def out_copy(i):
            b = i % _NBUF
            return pltpu.make_async_copy(
                buf.at[b],
                out_hbm.at[pl.ds(base + i * _CHUNK_ROWS, _CHUNK_ROWS)],
                sem_out.at[b],
            )

        in_copy(0).start()
        for i in range(num_chunks):
            if i + 1 < num_chunks:
                if i + 1 >= _NBUF:
                    out_copy(i + 1 - _NBUF).wait()
                in_copy(i + 1).start()
            in_copy(i).wait()
            out_copy(i).start()
        for i in range(max(0, num_chunks - _NBUF + 1), num_chunks):
            out_copy(i).wait()

    return copy_rows(table)[None]


# speedup vs baseline: 40.0713x; 2.3719x over previous
import functools

import jax
import jax.numpy as jnp
from jax.experimental import pallas as pl
from jax.experimental.pallas import tpu as pltpu

_BLOCK_ROWS = 512


def kernel(x, table):
    seq_len = x.shape[1]
    d_model = table.shape[1]

    def body(t_ref, o_ref):
        o_ref[...] = t_ref[...]

    out = pl.pallas_call(
        body,
        grid=(seq_len // _BLOCK_ROWS,),
        in_specs=[pl.BlockSpec((_BLOCK_ROWS, d_model), lambda i: (i, 0))],
        out_specs=pl.BlockSpec((_BLOCK_ROWS, d_model), lambda i: (i, 0)),
        out_shape=jax.ShapeDtypeStruct((seq_len, d_model), table.dtype),
    )(table)
    return out[None]
